# baseline (speedup 1.0000x reference)
import jax
import jax.numpy as jnp
from jax.experimental import pallas as pl
from jax.experimental.pallas import tpu as pltpu

_N = 4096
_D = 256
_T = 1024
_CH = 128


def _x3_kernel(zn_i_ref, zn_j_ref, out_ref, mx_ref):
    zn_i = zn_i_ref[...]
    acc = jnp.full((256, _CH), -2.0, jnp.float32)
    for c in range(_T // _CH):
        chunk = jax.lax.dot_general(
            zn_i, zn_j_ref[pl.ds(c * _CH, _CH), :],
            (((1,), (1,)), ((), ())),
            preferred_element_type=jnp.float32)
        m = jnp.maximum(jnp.maximum(chunk[0:256], chunk[256:512]),
                        jnp.maximum(chunk[512:768], chunk[768:1024]))
        acc = jnp.maximum(acc, m)
    mx_ref[0] = jnp.max(acc)
    out_ref[...] = jnp.zeros_like(out_ref)


def kernel(Z, planes):
    zn = Z.astype(jnp.bfloat16)
    out = pl.pallas_call(
        _x3_kernel,
        grid=(_N // _T, _N // _T),
        in_specs=[
            pl.BlockSpec((_T, _D), lambda i, j: (i, 0)),
            pl.BlockSpec((_T, _D), lambda i, j: (j, 0)),
        ],
        out_specs=pl.BlockSpec((_T, _T), lambda i, j: (i, j)),
        out_shape=jax.ShapeDtypeStruct((_N, _N), jnp.float32),
        scratch_shapes=[pltpu.SMEM((1,), jnp.float32)],
        compiler_params=pltpu.CompilerParams(
            dimension_semantics=("parallel", "parallel"),
        ),
    )(zn, zn)
    return out
